# baseline (speedup 1.0000x reference)
import functools

import jax
import jax.numpy as jnp
from jax import lax
from jax.experimental import pallas as pl
from jax.experimental.pallas import tpu as pltpu
from jax.experimental.pallas import tpu_sc as plsc


def _tanh_block(x_ref, o_ref):
    o_ref[...] = jnp.tanh(x_ref[...])


def _tc_tanh(x):
    rows, h, w = x.shape
    block = 24
    grid = rows // block
    return pl.pallas_call(
        _tanh_block,
        out_shape=jax.ShapeDtypeStruct((rows, h, w), jnp.float32),
        grid=(grid,),
        in_specs=[pl.BlockSpec((block, h, w), lambda i: (i, 0, 0))],
        out_specs=pl.BlockSpec((block, h, w), lambda i: (i, 0, 0)),
    )(x)


def _sc_tanh(x):
    rows, h, w = x.shape
    NC, NS = 2, 16
    NW = NC * NS
    per_w = rows // NW
    RCH = 96
    nch = h // RCH
    mesh = plsc.VectorSubcoreMesh(core_axis_name="c", subcore_axis_name="s")

    @functools.partial(
        pl.kernel,
        mesh=mesh,
        out_type=jax.ShapeDtypeStruct((rows, h, w), jnp.float32),
        scratch_types=[pltpu.VMEM((RCH, w), jnp.float32)],
    )
    def k(x_hbm, o_hbm, buf):
        wid = lax.axis_index("s") * NC + lax.axis_index("c")
        base = wid * per_w

        def img_body(ii, carry):
            i = base + ii

            def ch_body(c, carry2):
                r0 = c * RCH
                pltpu.sync_copy(x_hbm.at[i, pl.ds(r0, RCH)], buf)

                def row_body(r, carry3):
                    def col_body(j, carry4):
                        v = buf[r, pl.ds(j * 16, 16)]
                        t = jnp.exp(v * 2.0)
                        buf[r, pl.ds(j * 16, 16)] = 1.0 - 2.0 / (t + 1.0)
                        return carry4

                    return lax.fori_loop(0, w // 16, col_body, carry3)

                lax.fori_loop(0, RCH, row_body, 0)
                pltpu.sync_copy(buf, o_hbm.at[i, pl.ds(r0, RCH)])
                return carry2

            lax.fori_loop(0, nch, ch_body, 0)
            return carry

        lax.fori_loop(0, per_w, img_body, 0)

    return k(x)


def kernel(input, images):
    b = input.shape[0]
    n = images.shape[0]
    if b < n:
        images = images[:b]
    shape = images.shape
    h, w = shape[-2], shape[-1]
    rows = images.size // (h * w)
    x = images.reshape(rows, h, w)
    out = _sc_tanh(x)
    return out.reshape(shape)


# speedup vs baseline: 18.1068x; 18.1068x over previous
import jax
import jax.numpy as jnp
from jax.experimental import pallas as pl


def _tanh_block(x_ref, o_ref):
    o_ref[...] = jnp.tanh(x_ref[...])


def kernel(input, images):
    b = input.shape[0]
    n = images.shape[0]
    if b < n:
        images = images[:b]
    shape = images.shape
    h, w = shape[-2], shape[-1]
    rows = images.size // (h * w)
    block = 24
    grid = rows // block
    x = images.reshape(rows, h, w)
    out = pl.pallas_call(
        _tanh_block,
        out_shape=jax.ShapeDtypeStruct((rows, h, w), jnp.float32),
        grid=(grid,),
        in_specs=[pl.BlockSpec((block, h, w), lambda i: (i, 0, 0))],
        out_specs=pl.BlockSpec((block, h, w), lambda i: (i, 0, 0)),
    )(x)
    return out.reshape(shape)
